# Initial kernel scaffold; baseline (speedup 1.0000x reference)
import jax
import jax.numpy as jnp
from jax.experimental import pallas as pl


def kernel(x, pos_edge_index, neg_edge_index, W1, b1, W2, b2):
    raise NotImplementedError("write your pallas kernel here")


# speedup vs baseline: 3.1322x; 3.1322x over previous
import functools

import jax
import jax.numpy as jnp
from jax import lax
from jax.experimental import pallas as pl
from jax.experimental.pallas import tpu as pltpu
from jax.experimental.pallas import tpu_sc as plsc

_N = 10000
_D = 128
_L = 16
_NW = 32
_ROWS_TC = 1000


def _proj_body(x_ref, ma_ref, mb_ref, b1_ref, a_ref, b_ref):
    xv = x_ref[...]
    a_ref[...] = (
        jnp.dot(xv, ma_ref[...], preferred_element_type=jnp.float32) + b1_ref[...]
    )
    b_ref[...] = jnp.dot(xv, mb_ref[...], preferred_element_type=jnp.float32)


def _node_projections(x, ma, mb, b1row):
    grid = _N // _ROWS_TC
    return pl.pallas_call(
        _proj_body,
        grid=(grid,),
        in_specs=[
            pl.BlockSpec((_ROWS_TC, _D), lambda i: (i, 0)),
            pl.BlockSpec((_D, _D), lambda i: (0, 0)),
            pl.BlockSpec((_D, _D), lambda i: (0, 0)),
            pl.BlockSpec((1, _D), lambda i: (0, 0)),
        ],
        out_specs=[
            pl.BlockSpec((_ROWS_TC, _D), lambda i: (i, 0)),
            pl.BlockSpec((_ROWS_TC, _D), lambda i: (i, 0)),
        ],
        out_shape=[
            jax.ShapeDtypeStruct((_N, _D), jnp.float32),
            jax.ShapeDtypeStruct((_N, _D), jnp.float32),
        ],
    )(x, ma, mb, b1row)


def _sc_edge_kernel(num_edges, chunk):
    nchunk_total = num_edges // chunk
    assert nchunk_total % _NW == 0
    nchunk = nchunk_total // _NW
    epw = nchunk * chunk

    mesh = plsc.VectorSubcoreMesh(core_axis_name="c", subcore_axis_name="s")

    @functools.partial(
        pl.kernel,
        mesh=mesh,
        compiler_params=pltpu.CompilerParams(needs_layout_passes=False),
        out_type=jax.ShapeDtypeStruct((num_edges,), jnp.float32),
        scratch_types=[
            pltpu.VMEM((chunk,), jnp.int32),
            pltpu.VMEM((chunk,), jnp.int32),
            pltpu.VMEM((chunk, _D), jnp.float32),
            pltpu.VMEM((chunk, _D), jnp.float32),
            pltpu.VMEM((chunk,), jnp.float32),
            pltpu.VMEM((_L, _L), jnp.float32),
            pltpu.VMEM((_D,), jnp.float32),
            pltpu.VMEM((_L,), jnp.float32),
            pltpu.SemaphoreType.DMA,
            pltpu.SemaphoreType.DMA,
        ],
    )
    def k(a_hbm, b_hbm, src_hbm, tar_hbm, w2_hbm, b2_hbm, out_hbm,
          idxs_v, idxt_v, rows_a, rows_b, out_v, tr_v, w2_v, b2_v,
          sem_a, sem_b):
        wid = lax.axis_index("s") * 2 + lax.axis_index("c")
        base0 = wid * epw
        pltpu.sync_copy(w2_hbm, w2_v)
        pltpu.sync_copy(b2_hbm, b2_v)
        b2vec = b2_v[...]
        w2s = [w2_v[pl.ds(j * _L, _L)] for j in range(_D // _L)]
        iota16 = lax.iota(jnp.int32, _L)
        zero16 = jnp.zeros((_L,), jnp.float32)

        def chunk_body(g, carry):
            base = base0 + g * chunk
            pltpu.sync_copy(src_hbm.at[pl.ds(base, chunk)], idxs_v)
            pltpu.sync_copy(tar_hbm.at[pl.ds(base, chunk)], idxt_v)
            cp_a = pltpu.async_copy(a_hbm.at[idxs_v], rows_a, sem_a)
            cp_b = pltpu.async_copy(b_hbm.at[idxt_v], rows_b, sem_b)
            cp_a.wait()
            cp_b.wait()

            def grp_body(t, c):
                e0 = t * _L
                for i in range(_L):
                    e = e0 + i
                    acc = zero16
                    for j in range(_D // _L):
                        va = rows_a[e, pl.ds(j * _L, _L)]
                        vb = rows_b[e, pl.ds(j * _L, _L)]
                        acc = acc + jnp.maximum(va + vb, 0.0) * w2s[j]
                    tr_v[i, :] = acc
                outv = b2vec
                for kk in range(_L):
                    colk = plsc.load_gather(
                        tr_v, [iota16, jnp.full((_L,), kk, jnp.int32)])
                    outv = outv + colk
                out_v[pl.ds(e0, _L)] = outv
                return c

            lax.fori_loop(0, chunk // _L, grp_body, 0)
            pltpu.sync_copy(out_v, out_hbm.at[pl.ds(base, chunk)])
            return carry

        lax.fori_loop(0, nchunk, chunk_body, 0)

    return k


def kernel(x, pos_edge_index, neg_edge_index, W1, b1, W2, b2):
    num_edges = 2 * pos_edge_index.shape[1]
    fwd = jnp.concatenate([pos_edge_index, neg_edge_index], axis=-1)
    src = fwd[0]
    tar = fwd[1]

    ma = W1[:, :_D].T
    mb = W1[:, _D:].T
    b1row = b1.reshape(1, _D)
    a_tab, b_tab = _node_projections(x, ma, mb, b1row)

    w2 = W2.reshape(_D)
    b2v = jnp.full((_L,), b2[0], jnp.float32)

    out = _sc_edge_kernel(num_edges, chunk=80)(
        a_tab, b_tab, src, tar, w2, b2v)
    return out.reshape(num_edges, 1)


# speedup vs baseline: 6.1565x; 1.9656x over previous
import functools

import jax
import jax.numpy as jnp
from jax import lax
from jax.experimental import pallas as pl
from jax.experimental.pallas import tpu as pltpu
from jax.experimental.pallas import tpu_sc as plsc

_N = 10000
_D = 128
_L = 16
_NW = 32
_ROWS_TC = 1000


def _proj_body(x_ref, ma_ref, mb_ref, b1_ref, a_ref, b_ref):
    xv = x_ref[...]
    a_ref[...] = (
        jnp.dot(xv, ma_ref[...], preferred_element_type=jnp.float32) + b1_ref[...]
    )
    b_ref[...] = jnp.dot(xv, mb_ref[...], preferred_element_type=jnp.float32)


def _node_projections(x, ma, mb, b1row):
    grid = _N // _ROWS_TC
    return pl.pallas_call(
        _proj_body,
        grid=(grid,),
        in_specs=[
            pl.BlockSpec((_ROWS_TC, _D), lambda i: (i, 0)),
            pl.BlockSpec((_D, _D), lambda i: (0, 0)),
            pl.BlockSpec((_D, _D), lambda i: (0, 0)),
            pl.BlockSpec((1, _D), lambda i: (0, 0)),
        ],
        out_specs=[
            pl.BlockSpec((_ROWS_TC, _D), lambda i: (i, 0)),
            pl.BlockSpec((_ROWS_TC, _D), lambda i: (i, 0)),
        ],
        out_shape=[
            jax.ShapeDtypeStruct((_N, _D), jnp.float32),
            jax.ShapeDtypeStruct((_N, _D), jnp.float32),
        ],
    )(x, ma, mb, b1row)


def _sc_edge_kernel(num_edges, chunk):
    nchunk_total = num_edges // chunk
    assert nchunk_total % _NW == 0
    nchunk = nchunk_total // _NW
    epw = nchunk * chunk

    mesh = plsc.VectorSubcoreMesh(core_axis_name="c", subcore_axis_name="s")

    @functools.partial(
        pl.kernel,
        mesh=mesh,
        compiler_params=pltpu.CompilerParams(needs_layout_passes=False),
        out_type=jax.ShapeDtypeStruct((num_edges,), jnp.float32),
        scratch_types=[
            pltpu.VMEM((epw,), jnp.int32),
            pltpu.VMEM((epw,), jnp.int32),
            pltpu.VMEM((2, chunk, _D), jnp.float32),
            pltpu.VMEM((2, chunk, _D), jnp.float32),
            pltpu.VMEM((epw,), jnp.float32),
            pltpu.VMEM((_L, _L), jnp.float32),
            pltpu.VMEM((_D,), jnp.float32),
            pltpu.VMEM((_L,), jnp.float32),
            pltpu.SemaphoreType.DMA,
            pltpu.SemaphoreType.DMA,
            pltpu.SemaphoreType.DMA,
            pltpu.SemaphoreType.DMA,
        ],
    )
    def k(a_hbm, b_hbm, src_hbm, tar_hbm, w2_hbm, b2_hbm, out_hbm,
          idxs_v, idxt_v, rows_a, rows_b, out_v, tr_v, w2_v, b2_v,
          sem_a0, sem_a1, sem_b0, sem_b1):
        wid = lax.axis_index("s") * 2 + lax.axis_index("c")
        base0 = wid * epw
        pltpu.sync_copy(w2_hbm, w2_v)
        pltpu.sync_copy(b2_hbm, b2_v)
        pltpu.sync_copy(src_hbm.at[pl.ds(base0, epw)], idxs_v)
        pltpu.sync_copy(tar_hbm.at[pl.ds(base0, epw)], idxt_v)
        b2vec = b2_v[...]
        w2s = [w2_v[pl.ds(j * _L, _L)] for j in range(_D // _L)]
        iota16 = lax.iota(jnp.int32, _L)
        zero16 = jnp.zeros((_L,), jnp.float32)
        sems = [(sem_a0, sem_b0), (sem_a1, sem_b1)]

        def gather_descs(g, b):
            sa, sb = sems[b]
            idx_a = idxs_v.at[pl.ds(g * chunk, chunk)]
            idx_b = idxt_v.at[pl.ds(g * chunk, chunk)]
            return (pltpu.make_async_copy(a_hbm.at[idx_a], rows_a.at[b], sa),
                    pltpu.make_async_copy(b_hbm.at[idx_b], rows_b.at[b], sb))

        def start_gather(g, b):
            for cp in gather_descs(g, b):
                cp.start()

        def compute_chunk(g, b):
            for cp in gather_descs(g, b):
                cp.wait()

            def grp_body(t, c):
                e0 = t * _L
                for i in range(_L):
                    e = e0 + i
                    acc = zero16
                    for j in range(_D // _L):
                        va = rows_a[b, e, pl.ds(j * _L, _L)]
                        vb = rows_b[b, e, pl.ds(j * _L, _L)]
                        acc = acc + jnp.maximum(va + vb, 0.0) * w2s[j]
                    tr_v[i, :] = acc
                outv = b2vec
                for kk in range(_L):
                    colk = plsc.load_gather(
                        tr_v, [iota16, jnp.full((_L,), kk, jnp.int32)])
                    outv = outv + colk
                out_v[pl.ds(g * chunk + e0, _L)] = outv
                return c

            lax.fori_loop(0, chunk // _L, grp_body, 0)

        start_gather(0, 0)
        start_gather(1, 1)

        def pair_body(h, carry):
            for b in range(2):
                g = h * 2 + b
                compute_chunk(g, b)

                @pl.when(g + 2 < nchunk)
                def _():
                    start_gather(g + 2, b)
            return carry

        lax.fori_loop(0, nchunk // 2, pair_body, 0)
        if nchunk % 2:
            compute_chunk(nchunk - 1, (nchunk - 1) % 2)
        pltpu.sync_copy(out_v, out_hbm.at[pl.ds(base0, epw)])

    return k


def kernel(x, pos_edge_index, neg_edge_index, W1, b1, W2, b2):
    num_edges = 2 * pos_edge_index.shape[1]
    fwd = jnp.concatenate([pos_edge_index, neg_edge_index], axis=-1)
    src = fwd[0]
    tar = fwd[1]

    ma = W1[:, :_D].T
    mb = W1[:, _D:].T
    b1row = b1.reshape(1, _D)
    a_tab, b_tab = _node_projections(x, ma, mb, b1row)

    w2 = W2.reshape(_D)
    b2v = jnp.full((_L,), b2[0], jnp.float32)

    out = _sc_edge_kernel(num_edges, chunk=80)(
        a_tab, b_tab, src, tar, w2, b2v)
    return out.reshape(num_edges, 1)
